# Initial kernel scaffold; baseline (speedup 1.0000x reference)
import functools

import jax
import jax.numpy as jnp
from jax import lax
from jax.experimental import pallas as pl
from jax.experimental.pallas import tpu as pltpu
from jax.experimental.pallas import tpu_sc as plsc

N = 10000
E = 320000
D = 128
H = 128
G = 128

NC = 2
NS = 16
NW = NC * NS
CH = 128
NCHUNKS = E // CH
NPAD = 10240
RPT = NPAD // NS


def _zero_vmem_rows(ref, nrows):
    def body(i, _):
        r = i // 8
        c = (i % 8) * 16
        ref[r, pl.ds(c, 16)] = jnp.zeros((16,), jnp.float32)
        return _
    lax.fori_loop(0, nrows * 8, body, None)


def _zero_vmem_1d(ref, n):
    def body(i, _):
        ref[pl.ds(i * 16, 16)] = jnp.zeros((16,), jnp.float32)
        return _
    lax.fori_loop(0, n // 16, body, None)


def _make_sc_agg(with_counts):
    mesh = plsc.VectorSubcoreMesh(core_axis_name="c", subcore_axis_name="s")
    out_type = [jax.ShapeDtypeStruct((NC, NPAD, H), jnp.float32)]
    scratch = [
        pltpu.VMEM((CH,), jnp.int32),
        pltpu.VMEM((CH,), jnp.int32),
        pltpu.VMEM((CH, H), jnp.float32),
        pltpu.VMEM_SHARED((NPAD, H), jnp.float32),
        pltpu.SemaphoreType.DMA,
    ]
    if with_counts:
        out_type.append(jax.ShapeDtypeStruct((NC, NPAD), jnp.float32))
        scratch += [
            pltpu.VMEM((CH,), jnp.float32),
            pltpu.VMEM((RPT,), jnp.float32),
            pltpu.VMEM_SHARED((NPAD,), jnp.float32),
        ]

    def body(*refs):
        if with_counts:
            (p_hbm, src_hbm, dst_hbm, s_out, cnt_out,
             idx_s, idx_d, rows, acc, sem, ones, zcnt, cacc) = refs
        else:
            (p_hbm, src_hbm, dst_hbm, s_out,
             idx_s, idx_d, rows, acc, sem) = refs
        c = lax.axis_index("c")
        s = lax.axis_index("s")
        w = s * NC + c

        _zero_vmem_rows(rows, CH)
        for k in range(RPT // CH):
            pltpu.sync_copy(rows, acc.at[pl.ds(s * RPT + k * CH, CH)])
        if with_counts:
            def fill_ones(i, _):
                ones[pl.ds(i * 16, 16)] = jnp.full((16,), 1.0, jnp.float32)
                return _
            lax.fori_loop(0, CH // 16, fill_ones, None)
            _zero_vmem_1d(zcnt, RPT)
            pltpu.sync_copy(zcnt, cacc.at[pl.ds(s * RPT, RPT)])
        plsc.subcore_barrier()

        n_i = (NCHUNKS - w + NW - 1) // NW

        def edge_body(i, _):
            base = pl.multiple_of((w + i * NW) * CH, CH)
            pltpu.sync_copy(src_hbm.at[pl.ds(base, CH)], idx_s)
            pltpu.sync_copy(dst_hbm.at[pl.ds(base, CH)], idx_d)
            pltpu.async_copy(p_hbm.at[idx_s], rows, sem).wait()
            pltpu.sync_copy(rows, acc.at[idx_d], add=True)
            if with_counts:
                pltpu.sync_copy(ones, cacc.at[idx_d], add=True)
            return _
        lax.fori_loop(0, n_i, edge_body, None)
        plsc.subcore_barrier()

        for k in range(RPT // CH):
            r0 = s * RPT + k * CH
            pltpu.sync_copy(acc.at[pl.ds(r0, CH)], s_out.at[c, pl.ds(r0, CH)])
        if with_counts:
            pltpu.sync_copy(cacc.at[pl.ds(s * RPT, RPT)],
                            cnt_out.at[c, pl.ds(s * RPT, RPT)])

    return pl.kernel(body, mesh=mesh, out_type=out_type,
                     scratch_types=scratch)


_sc_agg_counts = _make_sc_agg(True)
_sc_agg = _make_sc_agg(False)


def _dot_t(a, w):
    return lax.dot_general(a, w, (((1,), (1,)), ((), ())),
                           preferred_element_type=jnp.float32)


def _tc_pre_body(x_ref, wl_ref, wr_ref, bl_ref, p_ref, q_ref):
    x = x_ref[...]
    p_ref[...] = _dot_t(x, wl_ref[...])
    q_ref[...] = _dot_t(x, wr_ref[...]) + bl_ref[...]


def _tc_pre(x, Wl, Wr, bl2):
    return pl.pallas_call(
        _tc_pre_body,
        out_shape=[jax.ShapeDtypeStruct((N, H), jnp.float32),
                   jax.ShapeDtypeStruct((N, H), jnp.float32)],
    )(x, Wl, Wr, bl2)


def _bn_relu(pre, g, b):
    m = jnp.mean(pre, axis=0, keepdims=True)
    v = jnp.mean((pre - m) ** 2, axis=0, keepdims=True)
    return jnp.maximum(g * (pre - m) * lax.rsqrt(v + 1e-5) + b, 0.0)


def _combine(s_ref, cnt_ref, q_ref):
    ssum = s_ref[0, :N, :] + s_ref[1, :N, :]
    cnt = cnt_ref[0, :N, :] + cnt_ref[1, :N, :]
    return ssum / jnp.maximum(cnt, 1.0) + q_ref[...]


def _tc_mid_body(s_ref, cnt_ref, q_ref, g_ref, b_ref, wl_ref, wr_ref, bl_ref,
                 p_ref, q2_ref):
    h = _bn_relu(_combine(s_ref, cnt_ref, q_ref), g_ref[...], b_ref[...])
    p_ref[...] = _dot_t(h, wl_ref[...])
    q2_ref[...] = _dot_t(h, wr_ref[...]) + bl_ref[...]


def _tc_mid(S, cnt3, Q, g2, b2, Wl, Wr, bl2):
    return pl.pallas_call(
        _tc_mid_body,
        out_shape=[jax.ShapeDtypeStruct((N, H), jnp.float32),
                   jax.ShapeDtypeStruct((N, H), jnp.float32)],
    )(S, cnt3, Q, g2, b2, Wl, Wr, bl2)


def _tc_post_body(s_ref, cnt_ref, q_ref, g_ref, b_ref, batch_ref, out_ref):
    h = _bn_relu(_combine(s_ref, cnt_ref, q_ref), g_ref[...], b_ref[...])
    gids = lax.broadcasted_iota(jnp.int32, (G, N), 0)
    onehot = (batch_ref[...] == gids).astype(jnp.float32)
    sums = lax.dot_general(onehot, h, (((1,), (0,)), ((), ())),
                           preferred_element_type=jnp.float32)
    cnts = jnp.sum(onehot, axis=1, keepdims=True)
    out_ref[...] = sums / jnp.maximum(cnts, 1.0)


def _tc_post(S, cnt3, Q, g2, b2, batch2):
    return pl.pallas_call(
        _tc_post_body,
        out_shape=jax.ShapeDtypeStruct((G, H), jnp.float32),
    )(S, cnt3, Q, g2, b2, batch2)


def kernel(x, edge_index, batch, Wl0, bl0, Wr0, Wl1, bl1, Wr1, g0, b0, g1, b1):
    src = edge_index[0]
    dst = edge_index[1]
    bl0_2 = bl0.reshape(1, H)
    bl1_2 = bl1.reshape(1, H)
    g0_2, b0_2 = g0.reshape(1, H), b0.reshape(1, H)
    g1_2, b1_2 = g1.reshape(1, H), b1.reshape(1, H)
    batch2 = batch.reshape(1, N)

    P0, Q0 = _tc_pre(x, Wl0, Wr0, bl0_2)
    S0, cnt = _sc_agg_counts(P0, src, dst)
    cnt3 = cnt.reshape(NC, NPAD, 1)
    P1, Q1 = _tc_mid(S0, cnt3, Q0, g0_2, b0_2, Wl1, Wr1, bl1_2)
    S1 = _sc_agg(P1, src, dst)
    out = _tc_post(S1, cnt3, Q1, g1_2, b1_2, batch2)
    return out


# speedup vs baseline: 6.8512x; 6.8512x over previous
import functools

import jax
import jax.numpy as jnp
from jax import lax
from jax.experimental import pallas as pl
from jax.experimental.pallas import tpu as pltpu
from jax.experimental.pallas import tpu_sc as plsc

N = 10000
E = 320000
D = 128
H = 128
G = 128

NC = 2
NS = 16
NW = NC * NS
CH = 128
NCHUNKS = E // CH
NPAD = 10240
RPT = NPAD // NS


def _zero_vmem_rows(ref, nrows):
    def body(i, _):
        r = i // 8
        c = (i % 8) * 16
        ref[r, pl.ds(c, 16)] = jnp.zeros((16,), jnp.float32)
        return _
    lax.fori_loop(0, nrows * 8, body, None)


def _zero_vmem_1d(ref, n):
    def body(i, _):
        ref[pl.ds(i * 16, 16)] = jnp.zeros((16,), jnp.float32)
        return _
    lax.fori_loop(0, n // 16, body, None)


def _make_sc_agg(with_counts):
    mesh = plsc.VectorSubcoreMesh(core_axis_name="c", subcore_axis_name="s")
    out_type = [jax.ShapeDtypeStruct((NC, NPAD, H), jnp.float32)]
    scratch = [
        pltpu.VMEM((CH,), jnp.int32),
        pltpu.VMEM((CH,), jnp.int32),
        pltpu.VMEM((CH, H), jnp.float32),
        pltpu.VMEM_SHARED((NPAD, H), jnp.float32),
        pltpu.SemaphoreType.DMA,
    ]
    if with_counts:
        out_type.append(jax.ShapeDtypeStruct((NC, NPAD), jnp.float32))
        scratch += [
            pltpu.VMEM((CH,), jnp.float32),
            pltpu.VMEM((RPT,), jnp.float32),
            pltpu.VMEM_SHARED((NPAD,), jnp.float32),
        ]

    def body(*refs):
        if with_counts:
            (p_hbm, src_hbm, dst_hbm, s_out, cnt_out,
             idx_s, idx_d, rows, acc, sem, ones, zcnt, cacc) = refs
        else:
            (p_hbm, src_hbm, dst_hbm, s_out,
             idx_s, idx_d, rows, acc, sem) = refs
        c = lax.axis_index("c")
        s = lax.axis_index("s")
        w = s * NC + c

        _zero_vmem_rows(rows, CH)
        for k in range(RPT // CH):
            pltpu.sync_copy(rows, acc.at[pl.ds(s * RPT + k * CH, CH)])
        if with_counts:
            def fill_ones(i, _):
                ones[pl.ds(i * 16, 16)] = jnp.full((16,), 1.0, jnp.float32)
                return _
            lax.fori_loop(0, CH // 16, fill_ones, None)
            _zero_vmem_1d(zcnt, RPT)
            pltpu.sync_copy(zcnt, cacc.at[pl.ds(s * RPT, RPT)])
        plsc.subcore_barrier()

        n_i = (NCHUNKS - w + NW - 1) // NW

        def edge_body(i, _):
            base = pl.multiple_of((w + i * NW) * CH, CH)
            pltpu.sync_copy(src_hbm.at[pl.ds(base, CH)], idx_s)
            pltpu.sync_copy(dst_hbm.at[pl.ds(base, CH)], idx_d)
            pltpu.async_copy(p_hbm.at[idx_s], rows, sem).wait()
            pltpu.sync_copy(rows, acc.at[idx_d], add=True)
            if with_counts:
                pltpu.sync_copy(ones, cacc.at[idx_d], add=True)
            return _
        lax.fori_loop(0, n_i, edge_body, None)
        plsc.subcore_barrier()

        for k in range(RPT // CH):
            r0 = s * RPT + k * CH
            pltpu.sync_copy(acc.at[pl.ds(r0, CH)], s_out.at[c, pl.ds(r0, CH)])
        if with_counts:
            pltpu.sync_copy(cacc.at[pl.ds(s * RPT, RPT)],
                            cnt_out.at[c, pl.ds(s * RPT, RPT)])

    return pl.kernel(body, mesh=mesh, out_type=out_type,
                     scratch_types=scratch)


_sc_agg_counts = _make_sc_agg(True)
_sc_agg = _make_sc_agg(False)


def _dot_t(a, w):
    return lax.dot_general(a, w, (((1,), (1,)), ((), ())),
                           preferred_element_type=jnp.float32)


def _tc_pre_body(x_ref, wl_ref, wr_ref, bl_ref, p_ref, q_ref):
    x = x_ref[...]
    p_ref[...] = _dot_t(x, wl_ref[...])
    q_ref[...] = _dot_t(x, wr_ref[...]) + bl_ref[...]


def _tc_pre(x, Wl, Wr, bl2):
    return pl.pallas_call(
        _tc_pre_body,
        out_shape=[jax.ShapeDtypeStruct((N, H), jnp.float32),
                   jax.ShapeDtypeStruct((N, H), jnp.float32)],
    )(x, Wl, Wr, bl2)


def _bn_relu(pre, g, b):
    m = jnp.mean(pre, axis=0, keepdims=True)
    v = jnp.mean((pre - m) ** 2, axis=0, keepdims=True)
    return jnp.maximum(g * (pre - m) * lax.rsqrt(v + 1e-5) + b, 0.0)


def _combine(s_ref, cnt_ref, q_ref):
    ssum = s_ref[0, :N, :] + s_ref[1, :N, :]
    cnt = cnt_ref[0, :N, :] + cnt_ref[1, :N, :]
    return ssum / jnp.maximum(cnt, 1.0) + q_ref[...]


def _tc_mid_body(s_ref, cnt_ref, q_ref, g_ref, b_ref, wl_ref, wr_ref, bl_ref,
                 p_ref, q2_ref):
    h = _bn_relu(_combine(s_ref, cnt_ref, q_ref), g_ref[...], b_ref[...])
    p_ref[...] = _dot_t(h, wl_ref[...])
    q2_ref[...] = _dot_t(h, wr_ref[...]) + bl_ref[...]


def _tc_mid(S, cnt3, Q, g2, b2, Wl, Wr, bl2):
    return pl.pallas_call(
        _tc_mid_body,
        out_shape=[jax.ShapeDtypeStruct((N, H), jnp.float32),
                   jax.ShapeDtypeStruct((N, H), jnp.float32)],
    )(S, cnt3, Q, g2, b2, Wl, Wr, bl2)


def _tc_post_body(s_ref, cnt_ref, q_ref, g_ref, b_ref, batch_ref, out_ref):
    h = _bn_relu(_combine(s_ref, cnt_ref, q_ref), g_ref[...], b_ref[...])
    gids = lax.broadcasted_iota(jnp.int32, (G, N), 0)
    onehot = (batch_ref[...] == gids).astype(jnp.float32)
    sums = lax.dot_general(onehot, h, (((1,), (0,)), ((), ())),
                           preferred_element_type=jnp.float32)
    cnts = jnp.sum(onehot, axis=1, keepdims=True)
    out_ref[...] = sums / jnp.maximum(cnts, 1.0)


def _tc_post(S, cnt3, Q, g2, b2, batch2):
    return pl.pallas_call(
        _tc_post_body,
        out_shape=jax.ShapeDtypeStruct((G, H), jnp.float32),
    )(S, cnt3, Q, g2, b2, batch2)


def kernel(x, edge_index, batch, Wl0, bl0, Wr0, Wl1, bl1, Wr1, g0, b0, g1, b1):
    src = edge_index[0]
    dst = edge_index[1]
    bl0_2 = bl0.reshape(1, H)
    bl1_2 = bl1.reshape(1, H)
    g0_2, b0_2 = g0.reshape(1, H), b0.reshape(1, H)
    g1_2, b1_2 = g1.reshape(1, H), b1.reshape(1, H)
    batch2 = batch.reshape(1, N)

    P0, Q0 = _tc_pre(x, Wl0, Wr0, bl0_2)
    S0, cnt = _sc_agg_counts(P0, src, dst)
    cnt3 = cnt.reshape(NC, NPAD, 1)
    P1, Q1 = _tc_mid(S0, cnt3, Q0, g0_2, b0_2, Wl1, Wr1, bl1_2)
    (S1,) = _sc_agg(P1, src, dst)
    out = _tc_post(S1, cnt3, Q1, g1_2, b1_2, batch2)
    return out
